# baseline (speedup 1.0000x reference)
import jax
import jax.numpy as jnp
from jax.experimental import pallas as pl
from jax.experimental.pallas import tpu as pltpu

_CHUNK = 10000
_N_CHUNKS = 10
_N_STRIPES = 10


def _encode_kernel(x_hbm, group_ref, hemi_ref, o_hbm, extra_ref, big_sem, small_sem):
    i = pl.program_id(0)
    n, d = x_hbm.shape
    stripe = n // _N_STRIPES

    def big_copy(s):
        return pltpu.make_async_copy(
            x_hbm.at[pl.ds(s * stripe, stripe), :],
            o_hbm.at[pl.ds(s * stripe, stripe), pl.ds(0, d)],
            big_sem.at[s],
        )

    @pl.when(i == 0)
    def _():
        for s in range(_N_STRIPES):
            big_copy(s).start()

    slot = jax.lax.rem(i, 2)

    def small_copy(step, sl):
        return pltpu.make_async_copy(
            extra_ref.at[sl],
            o_hbm.at[pl.ds(step * _CHUNK, _CHUNK), pl.ds(d, 4)],
            small_sem.at[sl],
        )

    @pl.when(i >= 2)
    def _():
        small_copy(i - 2, slot).wait()

    r0 = i * _CHUNK
    even = jnp.concatenate([group_ref[0:1, :], hemi_ref[0:1, :]], axis=1)
    odd = jnp.concatenate([group_ref[0:1, :], hemi_ref[1:2, :]], axis=1)
    rows = r0 + jax.lax.broadcasted_iota(jnp.int32, (_CHUNK, 4), 0)
    extra = jnp.where((rows & 1) == 1, odd, jnp.broadcast_to(even, (_CHUNK, 4)))

    @pl.when(r0 >= 7008)
    def _():
        extra_ref[slot] = extra

    @pl.when(r0 < 7008)
    def _():
        col = jax.lax.broadcasted_iota(jnp.int32, (_CHUNK, 4), 1)
        gid = jnp.where(rows % 1000 < 8, rows // 1000, 0)
        e = extra
        for g in range(1, 8):
            gval = jnp.concatenate([group_ref[g : g + 1, :]] * 2, axis=1)
            e = jnp.where((gid == g) & (col < 2), gval, e)
        extra_ref[slot] = e

    small_copy(i, slot).start()

    @pl.when(i == _N_CHUNKS - 1)
    def _():
        small_copy(i - 1, 1 - slot).wait()
        small_copy(i, slot).wait()
        for s in range(_N_STRIPES):
            big_copy(s).wait()


def kernel(x, edge_attr, group_emb, hemi_emb):
    n, d = x.shape
    x_out = pl.pallas_call(
        _encode_kernel,
        grid=(_N_CHUNKS,),
        in_specs=[
            pl.BlockSpec(memory_space=pltpu.MemorySpace.HBM),
            pl.BlockSpec(group_emb.shape, lambda i: (0, 0)),
            pl.BlockSpec(hemi_emb.shape, lambda i: (0, 0)),
        ],
        out_specs=pl.BlockSpec(memory_space=pltpu.MemorySpace.HBM),
        out_shape=jax.ShapeDtypeStruct((n, d + 4), x.dtype),
        scratch_shapes=[
            pltpu.VMEM((2, _CHUNK, 4), jnp.float32),
            pltpu.SemaphoreType.DMA((_N_STRIPES,)),
            pltpu.SemaphoreType.DMA((2,)),
        ],
    )(x, group_emb, hemi_emb)
    return (x_out, edge_attr.astype(jnp.float32))


# speedup vs baseline: 12.7349x; 12.7349x over previous
import jax
import jax.numpy as jnp
from jax.experimental import pallas as pl
from jax.experimental.pallas import tpu as pltpu

_BLOCK = 10000


def _encode_kernel(x_ref, group_ref, hemi_ref, o_ref):
    block = x_ref.shape[0]
    r0 = pl.program_id(0) * block
    o_ref[:, 0:128] = x_ref[...]

    even = jnp.concatenate([group_ref[0:1, :], hemi_ref[0:1, :]], axis=1)
    odd = jnp.concatenate([group_ref[0:1, :], hemi_ref[1:2, :]], axis=1)
    rows = r0 + jax.lax.broadcasted_iota(jnp.int32, (block, 4), 0)
    o_ref[:, 128:132] = jnp.where(
        (rows & 1) == 1, odd, jnp.broadcast_to(even, (block, 4))
    )

    for g in range(1, 8):
        gr = 1000 * g

        @pl.when((r0 <= gr) & (gr < r0 + block))
        def _(g=g, gr=gr):
            o_ref[pl.ds(gr - r0, 8), 128:130] = jnp.broadcast_to(
                group_ref[g : g + 1, :], (8, 2)
            )


def kernel(x, edge_attr, group_emb, hemi_emb):
    n, d = x.shape
    grid = n // _BLOCK
    x_out = pl.pallas_call(
        _encode_kernel,
        grid=(grid,),
        in_specs=[
            pl.BlockSpec((_BLOCK, d), lambda i: (i, 0)),
            pl.BlockSpec(group_emb.shape, lambda i: (0, 0)),
            pl.BlockSpec(hemi_emb.shape, lambda i: (0, 0)),
        ],
        out_specs=pl.BlockSpec((_BLOCK, d + 4), lambda i: (i, 0)),
        out_shape=jax.ShapeDtypeStruct((n, d + 4), x.dtype),
        compiler_params=pltpu.CompilerParams(
            dimension_semantics=("parallel",),
        ),
    )(x, group_emb, hemi_emb)
    return (x_out, edge_attr.astype(jnp.float32))
